# baseline (speedup 1.0000x reference)
import math

import jax
import jax.numpy as jnp
from jax import lax
from jax.experimental import pallas as pl
from jax.experimental.pallas import tpu as pltpu
from jax.experimental.pallas import tpu_sc as plsc

MUL = 32
N_NODES = 10000
N_EDGES = 320000
F = 4 * MUL
WF = 5 * MUL

NC = 2
NS = 16
NW = NC * NS
L = 16

EPW = N_EDGES // NW
C = 80
NCHUNK = EPW // C
G = C // L

RPT = 624
RTAIL_OFF = RPT * NS
RTAIL = N_NODES - RTAIL_OFF

PW0 = math.sqrt(0.5)
INV3 = 1.0 / math.sqrt(3.0)
INV6 = 1.0 / math.sqrt(6.0)
SQRT3 = math.sqrt(3.0)


def _sc_body(x_hbm, ev_hbm, w_hbm, src_hbm, dst_hbm, zero_hbm,
             out0, out1,
             acc, sh_buf,
             a_src, a_dst, a_ev, a_w, a_rows,
             b_src, b_dst, b_ev, b_w, b_rows,
             sa_src, sa_oth, sa_g, sa_sc,
             sb_src, sb_oth, sb_g, sb_sc):
    cid = lax.axis_index("c")
    sid = lax.axis_index("s")
    wid = cid * NS + sid
    base0 = wid * EPW

    setA = (a_src, a_dst, a_ev, a_w, a_rows, sa_src, sa_oth, sa_g, sa_sc)
    setB = (b_src, b_dst, b_ev, b_w, b_rows, sb_src, sb_oth, sb_g, sb_sc)

    roff = pl.multiple_of(sid * RPT, 8)
    pltpu.sync_copy(zero_hbm.at[pl.ds(roff, RPT)], acc.at[pl.ds(roff, RPT)])

    @pl.when(sid == 0)
    def _():
        pltpu.sync_copy(zero_hbm.at[pl.ds(RTAIL_OFF, RTAIL)],
                        acc.at[pl.ds(RTAIL_OFF, RTAIL)])

    plsc.subcore_barrier()

    iota = lax.iota(jnp.int32, L)

    def src_copy(c, S):
        s_v, _, _, _, _, s_src, _, _, _ = S
        base = base0 + c * C
        return pltpu.make_async_copy(src_hbm.at[pl.ds(base, C)], s_v, s_src)

    def oth_copies(c, S):
        _, d_v, e_v, w_v, _, _, s_oth, _, _ = S
        base = base0 + c * C
        return [
            pltpu.make_async_copy(dst_hbm.at[pl.ds(base, C)], d_v, s_oth),
            pltpu.make_async_copy(ev_hbm.at[pl.ds(3 * base, 3 * C)], e_v,
                                  s_oth),
            pltpu.make_async_copy(w_hbm.at[pl.ds(WF * base, WF * C)], w_v,
                                  s_oth),
        ]

    def gather_copy(S):
        s_v, _, _, _, rows, _, _, s_g, _ = S
        return pltpu.make_async_copy(x_hbm.at[s_v], rows, s_g)

    def scatter_copy(S):
        _, d_v, _, _, rows, _, _, _, s_sc = S
        return pltpu.make_async_copy(rows, acc.at[d_v], s_sc)

    def compute(S):
        _, _, ev_v, w_v, rows, _, _, _, _ = S

        def sh_group(g, carry2):
            ridx3 = 3 * (g * L + iota)
            vx = plsc.load_gather(ev_v, [ridx3])
            vy = plsc.load_gather(ev_v, [ridx3 + 1])
            vz = plsc.load_gather(ev_v, [ridx3 + 2])
            n2 = vx * vx + vy * vy + vz * vz
            bits = plsc.bitcast(n2, jnp.int32)
            y = plsc.bitcast(jnp.full((L,), 0x5F3759DF, jnp.int32)
                             - (bits >> 1), jnp.float32)
            for _ in range(3):
                y = y * (1.5 - 0.5 * n2 * y * y)
            s3 = SQRT3 * y
            off = g * L
            sh_buf[pl.ds(off, L)] = vx * s3
            sh_buf[pl.ds(C + off, L)] = vy * s3
            sh_buf[pl.ds(2 * C + off, L)] = vz * s3
            return carry2

        lax.fori_loop(0, G, sh_group, 0)

        iota3 = 3 * iota

        def tp_edge(e, carry2):
            shx = plsc.load_gather(sh_buf, [jnp.full((L,), e)])
            shy = plsc.load_gather(sh_buf, [jnp.full((L,), C + e)])
            shz = plsc.load_gather(sh_buf, [jnp.full((L,), 2 * C + e)])
            erow = jnp.full((L,), e)
            wb0 = e * WF
            xs = [(rows[e, pl.ds(h, L)],
                   rows[e, pl.ds(32 + h, L)],
                   rows[e, pl.ds(64 + h, L)],
                   rows[e, pl.ds(96 + h, L)],
                   w_v[pl.ds(wb0 + h, L)],
                   w_v[pl.ds(wb0 + MUL + h, L)],
                   w_v[pl.ds(wb0 + 2 * MUL + h, L)],
                   w_v[pl.ds(wb0 + 3 * MUL + h, L)],
                   w_v[pl.ds(wb0 + 4 * MUL + h, L)]) for h in (0, L)]
            outs = []
            for x0, xx, xy, xz, wa, wb, wc, wd, we in xs:
                dot = xx * shx + xy * shy + xz * shz
                m0 = PW0 * (wa * x0) + (PW0 * INV3) * (wd * dot)
                cx = xy * shz - xz * shy
                cy = xz * shx - xx * shz
                cz = xx * shy - xy * shx
                t0 = INV3 * (wb * x0)
                wcs = INV3 * wc
                wes = INV6 * we
                m1x = t0 * shx + wcs * xx + wes * cx
                m1y = t0 * shy + wcs * xy + wes * cy
                m1z = t0 * shz + wcs * xz + wes * cz
                outs.append((m0, m1x, m1y, m1z))
            for h, (m0, m1x, m1y, m1z) in zip((0, L), outs):
                rows[e, pl.ds(h, L)] = m0
                cbase = MUL + 3 * h + iota3
                plsc.store_scatter(rows, [erow, cbase], m1x)
                plsc.store_scatter(rows, [erow, cbase + 1], m1y)
                plsc.store_scatter(rows, [erow, cbase + 2], m1z)
            return carry2

        lax.fori_loop(0, C, tp_edge, 0)

    def half_body(c, S, T):
        gather_copy(S).wait()
        for d in oth_copies(c, S):
            d.wait()

        @pl.when(c + 2 < NCHUNK)
        def _():
            src_copy(c + 2, S).start()

        @pl.when(c >= 1)
        def _():
            scatter_copy(T).wait()

        @pl.when(c + 1 < NCHUNK)
        def _():
            for d in oth_copies(c + 1, T):
                d.start()
            src_copy(c + 1, T).wait()
            gather_copy(T).start()

        compute(S)
        _, d_v, _, _, rows, _, _, _, s_sc = S
        pltpu.async_copy(rows, acc.at[d_v], s_sc, add=True)

    src_copy(0, setA).start()
    for d in oth_copies(0, setA):
        d.start()
    src_copy(1, setB).start()
    src_copy(0, setA).wait()
    gather_copy(setA).start()

    def loop(k, carry):
        half_body(2 * k, setA, setB)

        @pl.when(2 * k + 1 < NCHUNK)
        def _():
            half_body(2 * k + 1, setB, setA)

        return carry

    lax.fori_loop(0, (NCHUNK + 1) // 2, loop, 0)

    scatter_copy(setA).wait()
    plsc.subcore_barrier()

    woff = pl.multiple_of(sid * RPT, 8)

    @pl.when(cid == 0)
    def _():
        pltpu.sync_copy(acc.at[pl.ds(woff, RPT)], out0.at[pl.ds(woff, RPT)])

        @pl.when(sid == 0)
        def _():
            pltpu.sync_copy(acc.at[pl.ds(RTAIL_OFF, RTAIL)],
                            out0.at[pl.ds(RTAIL_OFF, RTAIL)])

    @pl.when(cid == 1)
    def _():
        pltpu.sync_copy(acc.at[pl.ds(woff, RPT)], out1.at[pl.ds(woff, RPT)])

        @pl.when(sid == 0)
        def _():
            pltpu.sync_copy(acc.at[pl.ds(RTAIL_OFF, RTAIL)],
                            out1.at[pl.ds(RTAIL_OFF, RTAIL)])


def _tc_add(a, b):
    def body(a_ref, b_ref, o_ref):
        o_ref[...] = a_ref[...] + b_ref[...]

    blk = 1000
    return pl.pallas_call(
        body,
        out_shape=jax.ShapeDtypeStruct((N_NODES, F), jnp.float32),
        grid=(N_NODES // blk,),
        in_specs=[pl.BlockSpec((blk, F), lambda i: (i, 0)),
                  pl.BlockSpec((blk, F), lambda i: (i, 0))],
        out_specs=pl.BlockSpec((blk, F), lambda i: (i, 0)),
    )(a, b)


def kernel(x, edge_vec, weight, edge_src, edge_dst):
    x = jnp.concatenate([x[:, :MUL], x[:, MUL::3], x[:, MUL + 1::3],
                         x[:, MUL + 2::3]], axis=1)
    zeros = jnp.zeros_like(x)
    mesh = plsc.VectorSubcoreMesh(core_axis_name="c", subcore_axis_name="s",
                                  num_cores=NC, num_subcores=NS)
    sds = jax.ShapeDtypeStruct((N_NODES, F), jnp.float32)
    p0, p1 = pl.kernel(
        _sc_body,
        out_type=(sds, sds),
        mesh=mesh,
        compiler_params=pltpu.CompilerParams(needs_layout_passes=False),
        scratch_types=[
            pltpu.VMEM_SHARED((N_NODES, F), jnp.float32),
            pltpu.VMEM((3 * C,), jnp.float32),
            pltpu.VMEM((C,), jnp.int32),
            pltpu.VMEM((C,), jnp.int32),
            pltpu.VMEM((3 * C,), jnp.float32),
            pltpu.VMEM((WF * C,), jnp.float32),
            pltpu.VMEM((C, F), jnp.float32),
            pltpu.VMEM((C,), jnp.int32),
            pltpu.VMEM((C,), jnp.int32),
            pltpu.VMEM((3 * C,), jnp.float32),
            pltpu.VMEM((WF * C,), jnp.float32),
            pltpu.VMEM((C, F), jnp.float32),
            pltpu.SemaphoreType.DMA,
            pltpu.SemaphoreType.DMA,
            pltpu.SemaphoreType.DMA,
            pltpu.SemaphoreType.DMA,
            pltpu.SemaphoreType.DMA,
            pltpu.SemaphoreType.DMA,
            pltpu.SemaphoreType.DMA,
            pltpu.SemaphoreType.DMA,
        ],
    )(x, edge_vec.reshape(-1), weight.reshape(-1), edge_src, edge_dst, zeros)
    return _tc_add(p0, p1)


# speedup vs baseline: 1.1734x; 1.1734x over previous
import math

import jax
import jax.numpy as jnp
from jax import lax
from jax.experimental import pallas as pl
from jax.experimental.pallas import tpu as pltpu
from jax.experimental.pallas import tpu_sc as plsc

MUL = 32
N_NODES = 10000
N_EDGES = 320000
F = 4 * MUL
WF = 5 * MUL

NC = 2
NS = 16
NW = NC * NS
L = 16

EPW = N_EDGES // NW
C = 80
NCHUNK = EPW // C
G = C // L

RPT = 624
RTAIL_OFF = RPT * NS
RTAIL = N_NODES - RTAIL_OFF

PW0 = math.sqrt(0.5)
INV3 = 1.0 / math.sqrt(3.0)
INV6 = 1.0 / math.sqrt(6.0)
SQRT3 = math.sqrt(3.0)


def _sc_body(x_hbm, ev_hbm, w_hbm, src_hbm, dst_hbm, zero_hbm,
             out0, out1,
             acc, sh_buf, w_v,
             a_src, a_dst, a_ev, a_rows,
             b_src, b_dst, b_ev, b_rows,
             sa_src, sa_oth, sa_g, sa_sc,
             sb_src, sb_oth, sb_g, sb_sc, s_w):
    cid = lax.axis_index("c")
    sid = lax.axis_index("s")
    wid = cid * NS + sid
    base0 = wid * EPW

    setA = (a_src, a_dst, a_ev, a_rows, sa_src, sa_oth, sa_g, sa_sc)
    setB = (b_src, b_dst, b_ev, b_rows, sb_src, sb_oth, sb_g, sb_sc)

    roff = pl.multiple_of(sid * RPT, 8)
    pltpu.sync_copy(zero_hbm.at[pl.ds(roff, RPT)], acc.at[pl.ds(roff, RPT)])

    @pl.when(sid == 0)
    def _():
        pltpu.sync_copy(zero_hbm.at[pl.ds(RTAIL_OFF, RTAIL)],
                        acc.at[pl.ds(RTAIL_OFF, RTAIL)])

    plsc.subcore_barrier()

    iota = lax.iota(jnp.int32, L)

    def src_copy(c, S):
        s_v, _, _, _, s_src, _, _, _ = S
        base = base0 + c * C
        return pltpu.make_async_copy(src_hbm.at[pl.ds(base, C)], s_v, s_src)

    def oth_copies(c, S):
        _, d_v, e_v, _, _, s_oth, _, _ = S
        base = base0 + c * C
        return [
            pltpu.make_async_copy(dst_hbm.at[pl.ds(base, C)], d_v, s_oth),
            pltpu.make_async_copy(ev_hbm.at[pl.ds(3 * base, 3 * C)], e_v,
                                  s_oth),
        ]

    def w_copy(c):
        base = base0 + c * C
        return pltpu.make_async_copy(w_hbm.at[pl.ds(base, C)], w_v, s_w)

    def gather_copy(S):
        s_v, _, _, rows, _, _, s_g, _ = S
        return pltpu.make_async_copy(x_hbm.at[s_v], rows, s_g)

    def scatter_copy(S):
        _, d_v, _, rows, _, _, _, s_sc = S
        return pltpu.make_async_copy(rows, acc.at[d_v], s_sc)

    def compute(S):
        _, _, ev_v, rows, _, _, _, _ = S

        def sh_group(g, carry2):
            ridx3 = 3 * (g * L + iota)
            vx = plsc.load_gather(ev_v, [ridx3])
            vy = plsc.load_gather(ev_v, [ridx3 + 1])
            vz = plsc.load_gather(ev_v, [ridx3 + 2])
            n2 = vx * vx + vy * vy + vz * vz
            bits = plsc.bitcast(n2, jnp.int32)
            y = plsc.bitcast(jnp.full((L,), 0x5F3759DF, jnp.int32)
                             - (bits >> 1), jnp.float32)
            for _ in range(3):
                y = y * (1.5 - 0.5 * n2 * y * y)
            s3 = SQRT3 * y
            off = g * L
            sh_buf[pl.ds(off, L)] = vx * s3
            sh_buf[pl.ds(C + off, L)] = vy * s3
            sh_buf[pl.ds(2 * C + off, L)] = vz * s3
            return carry2

        lax.fori_loop(0, G, sh_group, 0)

        iota3 = 3 * iota

        def tp_edge(e, carry2):
            shx = plsc.load_gather(sh_buf, [jnp.full((L,), e)])
            shy = plsc.load_gather(sh_buf, [jnp.full((L,), C + e)])
            shz = plsc.load_gather(sh_buf, [jnp.full((L,), 2 * C + e)])
            erow = jnp.full((L,), e)
            xs = []
            for h in (0, L):
                cb = MUL + 3 * h + iota3
                xs.append((rows[e, pl.ds(h, L)],
                           plsc.load_gather(rows, [erow, cb]),
                           plsc.load_gather(rows, [erow, cb + 1]),
                           plsc.load_gather(rows, [erow, cb + 2]),
                           w_v[e, pl.ds(h, L)],
                           w_v[e, pl.ds(MUL + h, L)],
                           w_v[e, pl.ds(2 * MUL + h, L)],
                           w_v[e, pl.ds(3 * MUL + h, L)],
                           w_v[e, pl.ds(4 * MUL + h, L)]))
            outs = []
            for x0, xx, xy, xz, wa, wb, wc, wd, we in xs:
                dot = xx * shx + xy * shy + xz * shz
                m0 = PW0 * (wa * x0) + (PW0 * INV3) * (wd * dot)
                cx = xy * shz - xz * shy
                cy = xz * shx - xx * shz
                cz = xx * shy - xy * shx
                t0 = INV3 * (wb * x0)
                wcs = INV3 * wc
                wes = INV6 * we
                m1x = t0 * shx + wcs * xx + wes * cx
                m1y = t0 * shy + wcs * xy + wes * cy
                m1z = t0 * shz + wcs * xz + wes * cz
                outs.append((m0, m1x, m1y, m1z))
            for h, (m0, m1x, m1y, m1z) in zip((0, L), outs):
                rows[e, pl.ds(h, L)] = m0
                cbase = MUL + 3 * h + iota3
                plsc.store_scatter(rows, [erow, cbase], m1x)
                plsc.store_scatter(rows, [erow, cbase + 1], m1y)
                plsc.store_scatter(rows, [erow, cbase + 2], m1z)
            return carry2

        lax.fori_loop(0, C, tp_edge, 0)

    def half_body(c, S, T):
        gather_copy(S).wait()
        for d in oth_copies(c, S):
            d.wait()

        @pl.when(c + 2 < NCHUNK)
        def _():
            src_copy(c + 2, S).start()

        @pl.when(c >= 1)
        def _():
            scatter_copy(T).wait()

        @pl.when(c + 1 < NCHUNK)
        def _():
            for d in oth_copies(c + 1, T):
                d.start()
            src_copy(c + 1, T).wait()
            gather_copy(T).start()

        w_copy(c).wait()
        compute(S)

        @pl.when(c + 1 < NCHUNK)
        def _():
            w_copy(c + 1).start()

        _, d_v, _, rows, _, _, _, s_sc = S
        pltpu.async_copy(rows, acc.at[d_v], s_sc, add=True)

    src_copy(0, setA).start()
    for d in oth_copies(0, setA):
        d.start()
    w_copy(0).start()
    src_copy(1, setB).start()
    src_copy(0, setA).wait()
    gather_copy(setA).start()

    def loop(k, carry):
        half_body(2 * k, setA, setB)

        @pl.when(2 * k + 1 < NCHUNK)
        def _():
            half_body(2 * k + 1, setB, setA)

        return carry

    lax.fori_loop(0, (NCHUNK + 1) // 2, loop, 0)

    scatter_copy(setA).wait()
    plsc.subcore_barrier()

    woff = pl.multiple_of(sid * RPT, 8)

    @pl.when(cid == 0)
    def _():
        pltpu.sync_copy(acc.at[pl.ds(woff, RPT)], out0.at[pl.ds(woff, RPT)])

        @pl.when(sid == 0)
        def _():
            pltpu.sync_copy(acc.at[pl.ds(RTAIL_OFF, RTAIL)],
                            out0.at[pl.ds(RTAIL_OFF, RTAIL)])

    @pl.when(cid == 1)
    def _():
        pltpu.sync_copy(acc.at[pl.ds(woff, RPT)], out1.at[pl.ds(woff, RPT)])

        @pl.when(sid == 0)
        def _():
            pltpu.sync_copy(acc.at[pl.ds(RTAIL_OFF, RTAIL)],
                            out1.at[pl.ds(RTAIL_OFF, RTAIL)])


def _tc_add(a, b):
    def body(a_ref, b_ref, o_ref):
        o_ref[...] = a_ref[...] + b_ref[...]

    blk = 1000
    return pl.pallas_call(
        body,
        out_shape=jax.ShapeDtypeStruct((N_NODES, F), jnp.float32),
        grid=(N_NODES // blk,),
        in_specs=[pl.BlockSpec((blk, F), lambda i: (i, 0)),
                  pl.BlockSpec((blk, F), lambda i: (i, 0))],
        out_specs=pl.BlockSpec((blk, F), lambda i: (i, 0)),
    )(a, b)


def kernel(x, edge_vec, weight, edge_src, edge_dst):
    zeros = jnp.zeros_like(x)
    mesh = plsc.VectorSubcoreMesh(core_axis_name="c", subcore_axis_name="s",
                                  num_cores=NC, num_subcores=NS)
    sds = jax.ShapeDtypeStruct((N_NODES, F), jnp.float32)
    p0, p1 = pl.kernel(
        _sc_body,
        out_type=(sds, sds),
        mesh=mesh,
        compiler_params=pltpu.CompilerParams(needs_layout_passes=False),
        scratch_types=[
            pltpu.VMEM_SHARED((N_NODES, F), jnp.float32),
            pltpu.VMEM((3 * C,), jnp.float32),
            pltpu.VMEM((C, WF), jnp.float32),
            pltpu.VMEM((C,), jnp.int32),
            pltpu.VMEM((C,), jnp.int32),
            pltpu.VMEM((3 * C,), jnp.float32),
            pltpu.VMEM((C, F), jnp.float32),
            pltpu.VMEM((C,), jnp.int32),
            pltpu.VMEM((C,), jnp.int32),
            pltpu.VMEM((3 * C,), jnp.float32),
            pltpu.VMEM((C, F), jnp.float32),
            pltpu.SemaphoreType.DMA,
            pltpu.SemaphoreType.DMA,
            pltpu.SemaphoreType.DMA,
            pltpu.SemaphoreType.DMA,
            pltpu.SemaphoreType.DMA,
            pltpu.SemaphoreType.DMA,
            pltpu.SemaphoreType.DMA,
            pltpu.SemaphoreType.DMA,
            pltpu.SemaphoreType.DMA,
        ],
    )(x, edge_vec.reshape(-1), weight, edge_src, edge_dst, zeros)
    return _tc_add(p0, p1)


# speedup vs baseline: 1.4788x; 1.2603x over previous
import math

import jax
import jax.numpy as jnp
from jax import lax
from jax.experimental import pallas as pl
from jax.experimental.pallas import tpu as pltpu
from jax.experimental.pallas import tpu_sc as plsc

MUL = 32
N_NODES = 10000
N_EDGES = 320000
F = 4 * MUL
WF = 5 * MUL

NC = 2
NS = 16
NW = NC * NS
L = 16

EPW = N_EDGES // NW
C = 80
NCHUNK = EPW // C
G = C // L

RPT = 624
RTAIL_OFF = RPT * NS
RTAIL = N_NODES - RTAIL_OFF

PW0 = math.sqrt(0.5)
INV3 = 1.0 / math.sqrt(3.0)
INV6 = 1.0 / math.sqrt(6.0)
SQRT3 = math.sqrt(3.0)


def _sc_body(x_hbm, ev_hbm, w_hbm, src_hbm, dst_hbm, zero_hbm,
             out0, out1,
             acc, sh_buf, w_v,
             a_src, a_dst, a_ev, a_rows,
             b_src, b_dst, b_ev, b_rows,
             sa_src, sa_oth, sa_g, sa_sc,
             sb_src, sb_oth, sb_g, sb_sc, s_w):
    cid = lax.axis_index("c")
    sid = lax.axis_index("s")
    wid = cid * NS + sid
    base0 = wid * EPW

    setA = (a_src, a_dst, a_ev, a_rows, sa_src, sa_oth, sa_g, sa_sc)
    setB = (b_src, b_dst, b_ev, b_rows, sb_src, sb_oth, sb_g, sb_sc)

    roff = pl.multiple_of(sid * RPT, 8)
    pltpu.sync_copy(zero_hbm.at[pl.ds(roff, RPT)], acc.at[pl.ds(roff, RPT)])

    @pl.when(sid == 0)
    def _():
        pltpu.sync_copy(zero_hbm.at[pl.ds(RTAIL_OFF, RTAIL)],
                        acc.at[pl.ds(RTAIL_OFF, RTAIL)])

    plsc.subcore_barrier()

    iota = lax.iota(jnp.int32, L)
    iota3 = 3 * iota
    cbs = {h: MUL + 3 * h + iota3 for h in (0, L)}

    def src_copy(c, S):
        s_v, _, _, _, s_src, _, _, _ = S
        base = base0 + c * C
        return pltpu.make_async_copy(src_hbm.at[pl.ds(base, C)], s_v, s_src)

    def oth_copies(c, S):
        _, d_v, e_v, _, _, s_oth, _, _ = S
        base = base0 + c * C
        return [
            pltpu.make_async_copy(dst_hbm.at[pl.ds(base, C)], d_v, s_oth),
            pltpu.make_async_copy(ev_hbm.at[pl.ds(base, C)],
                                  e_v.at[pl.ds(0, C)], s_oth),
            pltpu.make_async_copy(ev_hbm.at[pl.ds(N_EDGES + base, C)],
                                  e_v.at[pl.ds(C, C)], s_oth),
            pltpu.make_async_copy(ev_hbm.at[pl.ds(2 * N_EDGES + base, C)],
                                  e_v.at[pl.ds(2 * C, C)], s_oth),
        ]

    def w_copy(c):
        base = base0 + c * C
        return pltpu.make_async_copy(w_hbm.at[pl.ds(base, C)], w_v, s_w)

    def gather_copy(S):
        s_v, _, _, rows, _, _, s_g, _ = S
        return pltpu.make_async_copy(x_hbm.at[s_v], rows, s_g)

    def scatter_copy(S):
        _, d_v, _, rows, _, _, _, s_sc = S
        return pltpu.make_async_copy(rows, acc.at[d_v], s_sc)

    def compute(S):
        _, _, ev_v, rows, _, _, _, _ = S

        def sh_group(g, carry2):
            off = g * L
            vx = ev_v[pl.ds(off, L)]
            vy = ev_v[pl.ds(C + off, L)]
            vz = ev_v[pl.ds(2 * C + off, L)]
            n2 = vx * vx + vy * vy + vz * vz
            bits = plsc.bitcast(n2, jnp.int32)
            y = plsc.bitcast(jnp.full((L,), 0x5F3759DF, jnp.int32)
                             - (bits >> 1), jnp.float32)
            for _ in range(3):
                y = y * (1.5 - 0.5 * n2 * y * y)
            s3 = SQRT3 * y
            sh_buf[pl.ds(off, L)] = vx * s3
            sh_buf[pl.ds(C + off, L)] = vy * s3
            sh_buf[pl.ds(2 * C + off, L)] = vz * s3
            return carry2

        lax.fori_loop(0, G, sh_group, 0)

        def tp_edge(e, carry2):
            shx = plsc.load_gather(sh_buf, [jnp.full((L,), e)])
            shy = plsc.load_gather(sh_buf, [jnp.full((L,), C + e)])
            shz = plsc.load_gather(sh_buf, [jnp.full((L,), 2 * C + e)])
            erow = jnp.full((L,), e)
            xs = []
            for h in (0, L):
                cb = cbs[h]
                xs.append((rows[e, pl.ds(h, L)],
                           plsc.load_gather(rows, [erow, cb]),
                           plsc.load_gather(rows, [erow, cb + 1]),
                           plsc.load_gather(rows, [erow, cb + 2]),
                           w_v[e, pl.ds(h, L)],
                           w_v[e, pl.ds(MUL + h, L)],
                           w_v[e, pl.ds(2 * MUL + h, L)],
                           w_v[e, pl.ds(3 * MUL + h, L)],
                           w_v[e, pl.ds(4 * MUL + h, L)]))
            outs = []
            for x0, xx, xy, xz, wa, wb, wc, wd, we in xs:
                dot = xx * shx + xy * shy + xz * shz
                m0 = PW0 * (wa * x0) + (PW0 * INV3) * (wd * dot)
                cx = xy * shz - xz * shy
                cy = xz * shx - xx * shz
                cz = xx * shy - xy * shx
                t0 = INV3 * (wb * x0)
                wcs = INV3 * wc
                wes = INV6 * we
                m1x = t0 * shx + wcs * xx + wes * cx
                m1y = t0 * shy + wcs * xy + wes * cy
                m1z = t0 * shz + wcs * xz + wes * cz
                outs.append((m0, m1x, m1y, m1z))
            for h, (m0, m1x, m1y, m1z) in zip((0, L), outs):
                rows[e, pl.ds(h, L)] = m0
                cbase = cbs[h]
                plsc.store_scatter(rows, [erow, cbase], m1x)
                plsc.store_scatter(rows, [erow, cbase + 1], m1y)
                plsc.store_scatter(rows, [erow, cbase + 2], m1z)
            return carry2

        lax.fori_loop(0, C, tp_edge, 0)

    def half_body(c, S, T):
        gather_copy(S).wait()
        for d in oth_copies(c, S):
            d.wait()

        @pl.when(c + 2 < NCHUNK)
        def _():
            src_copy(c + 2, S).start()

        @pl.when(c >= 1)
        def _():
            scatter_copy(T).wait()

        @pl.when(c + 1 < NCHUNK)
        def _():
            for d in oth_copies(c + 1, T):
                d.start()
            src_copy(c + 1, T).wait()
            gather_copy(T).start()

        w_copy(c).wait()
        compute(S)

        @pl.when(c + 1 < NCHUNK)
        def _():
            w_copy(c + 1).start()

        _, d_v, _, rows, _, _, _, s_sc = S
        pltpu.async_copy(rows, acc.at[d_v], s_sc, add=True)

    src_copy(0, setA).start()
    for d in oth_copies(0, setA):
        d.start()
    w_copy(0).start()
    src_copy(1, setB).start()
    src_copy(0, setA).wait()
    gather_copy(setA).start()

    def loop(k, carry):
        half_body(2 * k, setA, setB)

        @pl.when(2 * k + 1 < NCHUNK)
        def _():
            half_body(2 * k + 1, setB, setA)

        return carry

    lax.fori_loop(0, (NCHUNK + 1) // 2, loop, 0)

    scatter_copy(setA).wait()
    plsc.subcore_barrier()

    woff = pl.multiple_of(sid * RPT, 8)

    @pl.when(cid == 0)
    def _():
        pltpu.sync_copy(acc.at[pl.ds(woff, RPT)], out0.at[pl.ds(woff, RPT)])

        @pl.when(sid == 0)
        def _():
            pltpu.sync_copy(acc.at[pl.ds(RTAIL_OFF, RTAIL)],
                            out0.at[pl.ds(RTAIL_OFF, RTAIL)])

    @pl.when(cid == 1)
    def _():
        pltpu.sync_copy(acc.at[pl.ds(woff, RPT)], out1.at[pl.ds(woff, RPT)])

        @pl.when(sid == 0)
        def _():
            pltpu.sync_copy(acc.at[pl.ds(RTAIL_OFF, RTAIL)],
                            out1.at[pl.ds(RTAIL_OFF, RTAIL)])


def _tc_add(a, b):
    def body(a_ref, b_ref, o_ref):
        o_ref[...] = a_ref[...] + b_ref[...]

    blk = 1000
    return pl.pallas_call(
        body,
        out_shape=jax.ShapeDtypeStruct((N_NODES, F), jnp.float32),
        grid=(N_NODES // blk,),
        in_specs=[pl.BlockSpec((blk, F), lambda i: (i, 0)),
                  pl.BlockSpec((blk, F), lambda i: (i, 0))],
        out_specs=pl.BlockSpec((blk, F), lambda i: (i, 0)),
    )(a, b)


def kernel(x, edge_vec, weight, edge_src, edge_dst):
    zeros = jnp.zeros_like(x)
    mesh = plsc.VectorSubcoreMesh(core_axis_name="c", subcore_axis_name="s",
                                  num_cores=NC, num_subcores=NS)
    sds = jax.ShapeDtypeStruct((N_NODES, F), jnp.float32)
    p0, p1 = pl.kernel(
        _sc_body,
        out_type=(sds, sds),
        mesh=mesh,
        compiler_params=pltpu.CompilerParams(needs_layout_passes=False),
        scratch_types=[
            pltpu.VMEM_SHARED((N_NODES, F), jnp.float32),
            pltpu.VMEM((3 * C,), jnp.float32),
            pltpu.VMEM((C, WF), jnp.float32),
            pltpu.VMEM((C,), jnp.int32),
            pltpu.VMEM((C,), jnp.int32),
            pltpu.VMEM((3 * C,), jnp.float32),
            pltpu.VMEM((C, F), jnp.float32),
            pltpu.VMEM((C,), jnp.int32),
            pltpu.VMEM((C,), jnp.int32),
            pltpu.VMEM((3 * C,), jnp.float32),
            pltpu.VMEM((C, F), jnp.float32),
            pltpu.SemaphoreType.DMA,
            pltpu.SemaphoreType.DMA,
            pltpu.SemaphoreType.DMA,
            pltpu.SemaphoreType.DMA,
            pltpu.SemaphoreType.DMA,
            pltpu.SemaphoreType.DMA,
            pltpu.SemaphoreType.DMA,
            pltpu.SemaphoreType.DMA,
            pltpu.SemaphoreType.DMA,
        ],
    )(x, edge_vec.T.reshape(-1), weight, edge_src, edge_dst, zeros)
    return _tc_add(p0, p1)


# speedup vs baseline: 1.8687x; 1.2636x over previous
import math

import jax
import jax.numpy as jnp
from jax import lax
from jax.experimental import pallas as pl
from jax.experimental.pallas import tpu as pltpu
from jax.experimental.pallas import tpu_sc as plsc

MUL = 32
N_NODES = 10000
N_EDGES = 320000
F = 4 * MUL
WF = 5 * MUL

NC = 2
NS = 16
NW = NC * NS
L = 16

EPW = N_EDGES // NW
C = 80
C2 = C // 2
NCHUNK = EPW // C
G = C // L

RPT = 624
RTAIL_OFF = RPT * NS
RTAIL = N_NODES - RTAIL_OFF

PW0 = math.sqrt(0.5)
INV3 = 1.0 / math.sqrt(3.0)
INV6 = 1.0 / math.sqrt(6.0)
SQRT3 = math.sqrt(3.0)


def _sc_body(x_hbm, ev_hbm, w_hbm, src_hbm, dst_hbm, zero_hbm,
             out0, out1,
             acc, sh_buf, w1_v, w2_v,
             a_src, a_dst, a_ev, a_rows,
             b_src, b_dst, b_ev, b_rows,
             sa_src, sa_oth, sa_g, sa_sc,
             sb_src, sb_oth, sb_g, sb_sc, s_w1, s_w2):
    cid = lax.axis_index("c")
    sid = lax.axis_index("s")
    wid = cid * NS + sid
    base0 = wid * EPW

    setA = (a_src, a_dst, a_ev, a_rows, sa_src, sa_oth, sa_g, sa_sc)
    setB = (b_src, b_dst, b_ev, b_rows, sb_src, sb_oth, sb_g, sb_sc)

    roff = pl.multiple_of(sid * RPT, 8)
    pltpu.sync_copy(zero_hbm.at[pl.ds(roff, RPT)], acc.at[pl.ds(roff, RPT)])

    @pl.when(sid == 0)
    def _():
        pltpu.sync_copy(zero_hbm.at[pl.ds(RTAIL_OFF, RTAIL)],
                        acc.at[pl.ds(RTAIL_OFF, RTAIL)])

    plsc.subcore_barrier()

    iota = lax.iota(jnp.int32, L)
    iota3 = 3 * iota
    cbs = {h: MUL + 3 * h + iota3 for h in (0, L)}

    def src_copy(c, S):
        s_v, _, _, _, s_src, _, _, _ = S
        base = base0 + c * C
        return pltpu.make_async_copy(src_hbm.at[pl.ds(base, C)], s_v, s_src)

    def oth_copies(c, S):
        _, d_v, e_v, _, _, s_oth, _, _ = S
        base = base0 + c * C
        return [
            pltpu.make_async_copy(dst_hbm.at[pl.ds(base, C)], d_v, s_oth),
            pltpu.make_async_copy(ev_hbm.at[pl.ds(base, C)],
                                  e_v.at[pl.ds(0, C)], s_oth),
            pltpu.make_async_copy(ev_hbm.at[pl.ds(N_EDGES + base, C)],
                                  e_v.at[pl.ds(C, C)], s_oth),
            pltpu.make_async_copy(ev_hbm.at[pl.ds(2 * N_EDGES + base, C)],
                                  e_v.at[pl.ds(2 * C, C)], s_oth),
        ]

    def w1_copy(c):
        base = base0 + c * C
        return pltpu.make_async_copy(w_hbm.at[pl.ds(base, C2)], w1_v, s_w1)

    def w2_copy(c):
        base = base0 + c * C + C2
        return pltpu.make_async_copy(w_hbm.at[pl.ds(base, C2)], w2_v, s_w2)

    def gather_copy(S):
        s_v, _, _, rows, _, _, s_g, _ = S
        return pltpu.make_async_copy(x_hbm.at[s_v], rows, s_g)

    def scatter_copy(S):
        _, d_v, _, rows, _, _, _, s_sc = S
        return pltpu.make_async_copy(rows, acc.at[d_v], s_sc)

    def compute(S, c):
        _, _, ev_v, rows, _, _, _, _ = S

        def sh_group(g, carry2):
            off = g * L
            vx = ev_v[pl.ds(off, L)]
            vy = ev_v[pl.ds(C + off, L)]
            vz = ev_v[pl.ds(2 * C + off, L)]
            n2 = vx * vx + vy * vy + vz * vz
            bits = plsc.bitcast(n2, jnp.int32)
            y = plsc.bitcast(jnp.full((L,), 0x5F3759DF, jnp.int32)
                             - (bits >> 1), jnp.float32)
            for _ in range(3):
                y = y * (1.5 - 0.5 * n2 * y * y)
            s3 = SQRT3 * y
            sh_buf[pl.ds(off, L)] = vx * s3
            sh_buf[pl.ds(C + off, L)] = vy * s3
            sh_buf[pl.ds(2 * C + off, L)] = vz * s3
            return carry2

        lax.fori_loop(0, G, sh_group, 0)

        def tp_edge_for(w_v, off):
          def tp_edge(j, carry2):
            e = off + j
            shx = plsc.load_gather(sh_buf, [jnp.full((L,), e)])
            shy = plsc.load_gather(sh_buf, [jnp.full((L,), C + e)])
            shz = plsc.load_gather(sh_buf, [jnp.full((L,), 2 * C + e)])
            erow = jnp.full((L,), e)
            xs = []
            for h in (0, L):
                cb = cbs[h]
                xs.append((rows[e, pl.ds(h, L)],
                           plsc.load_gather(rows, [erow, cb]),
                           plsc.load_gather(rows, [erow, cb + 1]),
                           plsc.load_gather(rows, [erow, cb + 2]),
                           w_v[j, pl.ds(h, L)],
                           w_v[j, pl.ds(MUL + h, L)],
                           w_v[j, pl.ds(2 * MUL + h, L)],
                           w_v[j, pl.ds(3 * MUL + h, L)],
                           w_v[j, pl.ds(4 * MUL + h, L)]))
            outs = []
            for x0, xx, xy, xz, wa, wb, wc, wd, we in xs:
                dot = xx * shx + xy * shy + xz * shz
                m0 = PW0 * (wa * x0) + (PW0 * INV3) * (wd * dot)
                cx = xy * shz - xz * shy
                cy = xz * shx - xx * shz
                cz = xx * shy - xy * shx
                t0 = INV3 * (wb * x0)
                wcs = INV3 * wc
                wes = INV6 * we
                m1x = t0 * shx + wcs * xx + wes * cx
                m1y = t0 * shy + wcs * xy + wes * cy
                m1z = t0 * shz + wcs * xz + wes * cz
                outs.append((m0, m1x, m1y, m1z))
            for h, (m0, m1x, m1y, m1z) in zip((0, L), outs):
                rows[e, pl.ds(h, L)] = m0
                cbase = cbs[h]
                plsc.store_scatter(rows, [erow, cbase], m1x)
                plsc.store_scatter(rows, [erow, cbase + 1], m1y)
                plsc.store_scatter(rows, [erow, cbase + 2], m1z)
            return carry2
          return tp_edge

        w1_copy(c).wait()
        lax.fori_loop(0, C2, tp_edge_for(w1_v, 0), 0)

        @pl.when(c + 1 < NCHUNK)
        def _():
            w1_copy(c + 1).start()

        w2_copy(c).wait()
        lax.fori_loop(0, C2, tp_edge_for(w2_v, C2), 0)

        @pl.when(c + 1 < NCHUNK)
        def _():
            w2_copy(c + 1).start()

    def half_body(c, S, T):
        gather_copy(S).wait()
        for d in oth_copies(c, S):
            d.wait()

        @pl.when(c + 2 < NCHUNK)
        def _():
            src_copy(c + 2, S).start()

        @pl.when(c >= 1)
        def _():
            scatter_copy(T).wait()

        @pl.when(c + 1 < NCHUNK)
        def _():
            for d in oth_copies(c + 1, T):
                d.start()
            src_copy(c + 1, T).wait()
            gather_copy(T).start()

        compute(S, c)
        _, d_v, _, rows, _, _, _, s_sc = S
        pltpu.async_copy(rows, acc.at[d_v], s_sc, add=True)

    src_copy(0, setA).start()
    for d in oth_copies(0, setA):
        d.start()
    w1_copy(0).start()
    w2_copy(0).start()
    src_copy(1, setB).start()
    src_copy(0, setA).wait()
    gather_copy(setA).start()

    def loop(k, carry):
        half_body(2 * k, setA, setB)

        @pl.when(2 * k + 1 < NCHUNK)
        def _():
            half_body(2 * k + 1, setB, setA)

        return carry

    lax.fori_loop(0, (NCHUNK + 1) // 2, loop, 0)

    scatter_copy(setA).wait()
    plsc.subcore_barrier()

    woff = pl.multiple_of(sid * RPT, 8)

    @pl.when(cid == 0)
    def _():
        pltpu.sync_copy(acc.at[pl.ds(woff, RPT)], out0.at[pl.ds(woff, RPT)])

        @pl.when(sid == 0)
        def _():
            pltpu.sync_copy(acc.at[pl.ds(RTAIL_OFF, RTAIL)],
                            out0.at[pl.ds(RTAIL_OFF, RTAIL)])

    @pl.when(cid == 1)
    def _():
        pltpu.sync_copy(acc.at[pl.ds(woff, RPT)], out1.at[pl.ds(woff, RPT)])

        @pl.when(sid == 0)
        def _():
            pltpu.sync_copy(acc.at[pl.ds(RTAIL_OFF, RTAIL)],
                            out1.at[pl.ds(RTAIL_OFF, RTAIL)])


def _tc_add(a, b):
    def body(a_ref, b_ref, o_ref):
        o_ref[...] = a_ref[...] + b_ref[...]

    blk = 1000
    return pl.pallas_call(
        body,
        out_shape=jax.ShapeDtypeStruct((N_NODES, F), jnp.float32),
        grid=(N_NODES // blk,),
        in_specs=[pl.BlockSpec((blk, F), lambda i: (i, 0)),
                  pl.BlockSpec((blk, F), lambda i: (i, 0))],
        out_specs=pl.BlockSpec((blk, F), lambda i: (i, 0)),
    )(a, b)


def kernel(x, edge_vec, weight, edge_src, edge_dst):
    zeros = jnp.zeros_like(x)
    mesh = plsc.VectorSubcoreMesh(core_axis_name="c", subcore_axis_name="s",
                                  num_cores=NC, num_subcores=NS)
    sds = jax.ShapeDtypeStruct((N_NODES, F), jnp.float32)
    p0, p1 = pl.kernel(
        _sc_body,
        out_type=(sds, sds),
        mesh=mesh,
        compiler_params=pltpu.CompilerParams(needs_layout_passes=False),
        scratch_types=[
            pltpu.VMEM_SHARED((N_NODES, F), jnp.float32),
            pltpu.VMEM((3 * C,), jnp.float32),
            pltpu.VMEM((C2, WF), jnp.float32),
            pltpu.VMEM((C2, WF), jnp.float32),
            pltpu.VMEM((C,), jnp.int32),
            pltpu.VMEM((C,), jnp.int32),
            pltpu.VMEM((3 * C,), jnp.float32),
            pltpu.VMEM((C, F), jnp.float32),
            pltpu.VMEM((C,), jnp.int32),
            pltpu.VMEM((C,), jnp.int32),
            pltpu.VMEM((3 * C,), jnp.float32),
            pltpu.VMEM((C, F), jnp.float32),
            pltpu.SemaphoreType.DMA,
            pltpu.SemaphoreType.DMA,
            pltpu.SemaphoreType.DMA,
            pltpu.SemaphoreType.DMA,
            pltpu.SemaphoreType.DMA,
            pltpu.SemaphoreType.DMA,
            pltpu.SemaphoreType.DMA,
            pltpu.SemaphoreType.DMA,
            pltpu.SemaphoreType.DMA,
            pltpu.SemaphoreType.DMA,
        ],
    )(x, edge_vec.T.reshape(-1), weight, edge_src, edge_dst, zeros)
    return _tc_add(p0, p1)
